# baseline (speedup 1.0000x reference)
import functools

import jax
import jax.numpy as jnp
from jax import lax
from jax.experimental import pallas as pl
from jax.experimental.pallas import tpu as pltpu
from jax.experimental.pallas import tpu_sc as plsc

_N = 10000
_E = 160000
_D = 256
_C = 256
_HD = 128

_R = 512
_GRID = 20
_NP = _R * _GRID

_NTILE = 16
_NCHUNK = 79
_EPT = _NCHUNK * 128
_EPAD = _EPT * _NTILE
_DENP = 10240


def _tc_layer0(x_ref, wcat_ref, skipb_ref, atts_ref, attd_ref,
               hsil_ref, skip_ref, as_ref, ad_ref):
    xb = x_ref[...]
    res = jnp.dot(xb, wcat_ref[...], preferred_element_type=jnp.float32)
    hs = res[:, :_C]
    hd = res[:, _C:2 * _C]
    sk = res[:, 2 * _C:] + skipb_ref[...]
    hsil_ref[...] = hs.reshape(_R, 2, 128).reshape(2 * _R, 128)
    skip_ref[...] = jnp.stack([sk[:, :128], sk[:, 128:]], axis=0)
    as_ref[...] = jnp.sum(hs * atts_ref[...], axis=1).reshape(1, 1, _R)
    ad_ref[...] = jnp.sum(hd * attd_ref[...], axis=1).reshape(1, 1, _R)


def _tc_layer1(agg_ref, skip_ref, wcat_ref, skipb_ref, atts_ref, attd_ref,
               hpil_ref, skip1_ref, as_ref, ad_ref):
    h2 = jax.nn.relu(agg_ref[...] + skip_ref[...])
    h = jnp.concatenate([h2[0], h2[1]], axis=1)
    res = jnp.dot(h, wcat_ref[...], preferred_element_type=jnp.float32)
    hp = res[:, :_C]
    sk1 = res[:, _C:] + skipb_ref[...]
    hpil_ref[...] = hp.reshape(_R, 2, 128).reshape(2 * _R, 128)
    skip1_ref[...] = jnp.stack([sk1[:, :128], sk1[:, 128:]], axis=0)
    as_ref[...] = jnp.sum(hp * atts_ref[...], axis=1).reshape(1, 1, _R)
    ad_ref[...] = jnp.sum(hp * attd_ref[...], axis=1).reshape(1, 1, _R)


def _tc_head(agg_ref, skip_ref, hw_ref, hb_ref, out_ref):
    h2 = agg_ref[...] + skip_ref[...]
    h = jnp.concatenate([h2[0], h2[1]], axis=1)
    out_ref[...] = (
        jnp.dot(h, hw_ref[...], preferred_element_type=jnp.float32)
        + hb_ref[...])


_f32 = jnp.float32

_layer0_call = pl.pallas_call(
    _tc_layer0,
    grid=(_GRID,),
    in_specs=[
        pl.BlockSpec((_R, _D), lambda i: (i, 0)),
        pl.BlockSpec((_D, 3 * _C), lambda i: (0, 0)),
        pl.BlockSpec((1, _C), lambda i: (0, 0)),
        pl.BlockSpec((1, _C), lambda i: (0, 0)),
        pl.BlockSpec((1, _C), lambda i: (0, 0)),
    ],
    out_specs=[
        pl.BlockSpec((2 * _R, 128), lambda i: (i, 0)),
        pl.BlockSpec((2, _R, 128), lambda i: (0, i, 0)),
        pl.BlockSpec((1, 1, _R), lambda i: (i, 0, 0)),
        pl.BlockSpec((1, 1, _R), lambda i: (i, 0, 0)),
    ],
    out_shape=[
        jax.ShapeDtypeStruct((2 * _NP, 128), _f32),
        jax.ShapeDtypeStruct((2, _NP, 128), _f32),
        jax.ShapeDtypeStruct((_GRID, 1, _R), _f32),
        jax.ShapeDtypeStruct((_GRID, 1, _R), _f32),
    ],
)

_layer1_call = pl.pallas_call(
    _tc_layer1,
    grid=(_GRID,),
    in_specs=[
        pl.BlockSpec((2, _R, 128), lambda i: (0, i, 0)),
        pl.BlockSpec((2, _R, 128), lambda i: (0, i, 0)),
        pl.BlockSpec((_C, 2 * _C), lambda i: (0, 0)),
        pl.BlockSpec((1, _C), lambda i: (0, 0)),
        pl.BlockSpec((1, _C), lambda i: (0, 0)),
        pl.BlockSpec((1, _C), lambda i: (0, 0)),
    ],
    out_specs=[
        pl.BlockSpec((2 * _R, 128), lambda i: (i, 0)),
        pl.BlockSpec((2, _R, 128), lambda i: (0, i, 0)),
        pl.BlockSpec((1, 1, _R), lambda i: (i, 0, 0)),
        pl.BlockSpec((1, 1, _R), lambda i: (i, 0, 0)),
    ],
    out_shape=[
        jax.ShapeDtypeStruct((2 * _NP, 128), _f32),
        jax.ShapeDtypeStruct((2, _NP, 128), _f32),
        jax.ShapeDtypeStruct((_GRID, 1, _R), _f32),
        jax.ShapeDtypeStruct((_GRID, 1, _R), _f32),
    ],
)

_head_call = pl.pallas_call(
    _tc_head,
    grid=(_GRID,),
    in_specs=[
        pl.BlockSpec((2, _R, 128), lambda i: (0, i, 0)),
        pl.BlockSpec((2, _R, 128), lambda i: (0, i, 0)),
        pl.BlockSpec((_C, _HD), lambda i: (0, 0)),
        pl.BlockSpec((1, _HD), lambda i: (0, 0)),
    ],
    out_specs=pl.BlockSpec((_R, _HD), lambda i: (i, 0)),
    out_shape=jax.ShapeDtypeStruct((_NP, _HD), _f32),
)


_mesh = plsc.VectorSubcoreMesh(core_axis_name="c", subcore_axis_name="s")

_NH = 5000
_NACC = 5008


@functools.partial(
    pl.kernel,
    out_type=jax.ShapeDtypeStruct((2, _NP, 128), _f32),
    mesh=_mesh,
    compiler_params=pltpu.CompilerParams(needs_layout_passes=False),
    scratch_types=[
        pltpu.VMEM((_NCHUNK, 128), jnp.int32),
        pltpu.VMEM((_NCHUNK, 128), jnp.int32),
        pltpu.VMEM((_NCHUNK, 128), jnp.int32),
        pltpu.VMEM((_NCHUNK, 128), _f32),
        pltpu.VMEM((_NCHUNK, 128), _f32),
        pltpu.VMEM((640,), _f32),
        pltpu.VMEM((128, 128), _f32),
        pltpu.VMEM((128, 128), _f32),
        pltpu.VMEM((128,), _f32),
        pltpu.VMEM((128,), _f32),
        pltpu.VMEM((128,), _f32),
        pltpu.VMEM((128,), _f32),
        pltpu.VMEM((128,), jnp.int32),
        pltpu.VMEM((128,), jnp.int32),
        pltpu.VMEM((128,), jnp.int32),
        pltpu.VMEM((128,), jnp.int32),
        pltpu.SemaphoreType.DMA,
        pltpu.SemaphoreType.DMA,
        pltpu.SemaphoreType.DMA,
        pltpu.SemaphoreType.DMA,
        pltpu.SemaphoreType.DMA,
        pltpu.SemaphoreType.DMA,
        pltpu.SemaphoreType.DMA,
        pltpu.SemaphoreType.DMA,
        pltpu.SemaphoreType.DMA,
        pltpu.VMEM_SHARED((_NACC, 128), _f32),
        pltpu.VMEM_SHARED((_DENP,), _f32),
    ],
)
def _gat_edges(hs_hbm, as_hbm, ad_hbm, src_hbm, dst_hbm, out_hbm,
               src_v, dst_v, li1_v, ex0_v, ex1_v, den_v, rows_a, rows_b,
               asg_a, asg_b, adg_a, adg_b, gst_a, gst_b, sst_a, sst_b,
               sem_ra, sem_rb, sem_sa, sem_sb, sem_da, sem_db, sem_dn,
               sem_wa, sem_wb, num_sh, den_sh):
    c = lax.axis_index("c")
    s = lax.axis_index("s")

    pltpu.sync_copy(src_hbm.at[s], src_v)
    pltpu.sync_copy(dst_hbm.at[s], dst_v)

    zs_f = jnp.zeros((16,), _f32)
    padp = jnp.full((16,), _NH << 15, jnp.int32)
    ebase = s * _EPT
    nch = jnp.where(s == _NTILE - 1, 5, 8)

    def _zero_den(k, _):
        den_v[pl.ds(k * 16, 16)] = zs_f
        return _
    lax.fori_loop(0, 40, _zero_den, None)
    pltpu.sync_copy(den_v, den_sh.at[pl.ds(s * 640, 640)])

    def _zero_acc():
        def _zero_row(r, _):
            for q in range(8):
                rows_a[r, pl.ds(q * 16, 16)] = zs_f
            return _
        lax.fori_loop(0, 40, _zero_row, None)

        def _zero_num(m, _):
            pltpu.sync_copy(rows_a.at[pl.ds(0, 40)],
                            num_sh.at[pl.ds(s * 320 + m * 40, 40)])
            return _
        lax.fori_loop(0, nch, _zero_num, None)

    _zero_acc()
    plsc.subcore_barrier()

    def _issue_ag(j, asg, adg, sem_s, sem_d):
        pltpu.async_copy(as_hbm.at[src_v.at[j]], asg, sem_s)
        pltpu.async_copy(ad_hbm.at[dst_v.at[j]], adg, sem_d)

    def _scan_chunk(j, cnts, asg, adg, sem_s, sem_d):
        pltpu.make_async_copy(as_hbm.at[src_v.at[j]], asg, sem_s).wait()
        pltpu.make_async_copy(ad_hbm.at[dst_v.at[j]], adg, sem_d).wait()

        def _grp(k, cnts):
            cnt0, cnt1 = cnts
            off = k * 16
            sl = pl.ds(off, 16)
            s16 = src_v[j, sl]
            d16 = dst_v[j, sl]
            al = asg[sl] + adg[sl]
            al = jnp.where(al >= 0.0, al, al * jnp.float32(0.2))
            ex = jnp.exp(al)
            gid = ebase + j * 128 + off + lax.iota(jnp.int32, 16)
            ex = jnp.where(gid < _E, ex, jnp.float32(0.0))
            rows_a[j, sl] = ex
            gi = s16 * 2 + c
            own0 = d16 < _NH
            i0 = own0.astype(jnp.int32)
            pos0 = cnt0 + plsc.cumsum(i0) - 1
            p0 = gi | (d16 << 15)
            plsc.store_scatter(src_v, [pos0 >> 7, pos0 & 127], p0, mask=own0)
            plsc.store_scatter(ex0_v, [pos0 >> 7, pos0 & 127], ex, mask=own0)
            own1 = jnp.logical_not(own0)
            pos1 = cnt1 + plsc.cumsum(1 - i0) - 1
            p1 = gi | ((d16 - _NH) << 15)
            plsc.store_scatter(li1_v, [pos1 >> 7, pos1 & 127], p1, mask=own1)
            plsc.store_scatter(ex1_v, [pos1 >> 7, pos1 & 127], ex, mask=own1)
            n0 = jnp.sum(i0)
            return (cnt0 + n0, cnt1 + (16 - n0))

        cnts = lax.fori_loop(0, 8, _grp, cnts)
        pltpu.async_copy(rows_a.at[j], den_sh.at[dst_v.at[j]], sem_dn,
                         add=True)
        return cnts

    _issue_ag(0, asg_a, adg_a, sem_sa, sem_da)

    def _scan_pair(pp, cnts):
        j0 = pp * 2
        _issue_ag(j0 + 1, asg_b, adg_b, sem_sb, sem_db)
        cnts = _scan_chunk(j0, cnts, asg_a, adg_a, sem_sa, sem_da)
        _issue_ag(j0 + 2, asg_a, adg_a, sem_sa, sem_da)
        cnts = _scan_chunk(j0 + 1, cnts, asg_b, adg_b, sem_sb, sem_db)
        return cnts

    cnts = lax.fori_loop(0, (_NCHUNK - 1) // 2, _scan_pair,
                         (jnp.int32(0), jnp.int32(0)))
    cnt0, cnt1 = _scan_chunk(_NCHUNK - 1, cnts, asg_a, adg_a, sem_sa, sem_da)

    def _drain_dn(j, _):
        pltpu.make_async_copy(rows_a.at[j], den_sh.at[dst_v.at[j]],
                              sem_dn).wait()
        return _
    lax.fori_loop(0, _NCHUNK, _drain_dn, None)

    def _pad_tail(cnt, li_ref, ex_ref):
        end = ((cnt + 127) >> 7) << 7
        for k in range(8):
            t = cnt + k * 16 + lax.iota(jnp.int32, 16)
            m = t < end
            rr = t >> 7
            cc = t & 127
            plsc.store_scatter(li_ref, [rr, cc], padp, mask=m)
            plsc.store_scatter(ex_ref, [rr, cc], zs_f, mask=m)
        return (cnt + 127) >> 7

    nchunk0 = _pad_tail(cnt0, src_v, ex0_v)
    nchunk1 = _pad_tail(cnt1, li1_v, ex1_v)

    def _rows_stage(nchunks, li_ref, ex_ref):
        def _unpack(jj, gst, sst):
            def _u(k, _):
                sl = pl.ds(k * 16, 16)
                p = li_ref[jj, sl]
                gst[sl] = p & 32767
                sst[sl] = lax.shift_right_logical(p, 15)
                return _
            lax.fori_loop(0, 8, _u, None)

        def _do(jj, rows_ref, gst, sst, sem, sem_w):
            pltpu.make_async_copy(hs_hbm.at[gst], rows_ref, sem).wait()

            def _scale(r, _):
                e = plsc.load_gather(
                    ex_ref, [jnp.full((16,), jj, jnp.int32),
                             jnp.full((16,), r, jnp.int32)])
                for q in range(8):
                    sl = pl.ds(q * 16, 16)
                    rows_ref[r, sl] = rows_ref[r, sl] * e
                return _
            lax.fori_loop(0, 128, _scale, None, unroll=4)
            pltpu.async_copy(rows_ref, num_sh.at[sst], sem_w, add=True)

        @pl.when(nchunks > 0)
        def _():
            _unpack(0, gst_a, sst_a)
            pltpu.async_copy(hs_hbm.at[gst_a], rows_a, sem_ra)

        def _pairr(pp, _):
            j0 = pp * 2
            j1 = j0 + 1

            @pl.when(j1 < nchunks)
            def _():
                @pl.when(pp > 0)
                def _():
                    pltpu.make_async_copy(
                        rows_b, num_sh.at[sst_b], sem_wb).wait()
                _unpack(j1, gst_b, sst_b)
                pltpu.async_copy(hs_hbm.at[gst_b], rows_b, sem_rb)
            _do(j0, rows_a, gst_a, sst_a, sem_ra, sem_wa)

            @pl.when(j1 < nchunks)
            def _():
                _do(j1, rows_b, gst_b, sst_b, sem_rb, sem_wb)

                @pl.when(j1 + 1 < nchunks)
                def _():
                    pltpu.make_async_copy(
                        rows_a, num_sh.at[sst_a], sem_wa).wait()
                    _unpack(j1 + 1, gst_a, sst_a)
                    pltpu.async_copy(hs_hbm.at[gst_a], rows_a, sem_ra)
            return _
        lax.fori_loop(0, (nchunks + 1) >> 1, _pairr, None)

        @pl.when(nchunks > 0)
        def _():
            pltpu.make_async_copy(rows_a, num_sh.at[sst_a], sem_wa).wait()

        @pl.when(nchunks > 1)
        def _():
            pltpu.make_async_copy(rows_b, num_sh.at[sst_b], sem_wb).wait()

    def _norm(nbase, dbase):
        def _body(m, _):
            l0 = s * 320 + m * 40
            pltpu.sync_copy(num_sh.at[pl.ds(l0, 40)], rows_a.at[pl.ds(0, 40)])

            def _scale_out(r, _):
                rc = plsc.load_gather(
                    den_v, [jnp.full((16,), dbase + m * 40 + r, jnp.int32)])
                for q in range(8):
                    sl = pl.ds(q * 16, 16)
                    rows_a[r, sl] = rows_a[r, sl] * rc
                return _
            lax.fori_loop(0, 40, _scale_out, None, unroll=4)
            pltpu.sync_copy(rows_a.at[pl.ds(0, 40)],
                            out_hbm.at[c, pl.ds(nbase + l0, 40)])
            return _
        lax.fori_loop(0, nch, _body, None)

    _rows_stage(nchunk0, src_v, ex0_v)
    plsc.subcore_barrier()

    pltpu.sync_copy(den_sh.at[pl.ds(s * 320, 320)], den_v.at[pl.ds(0, 320)])
    pltpu.sync_copy(den_sh.at[pl.ds(_NH + s * 320, 320)],
                    den_v.at[pl.ds(320, 320)])

    def _rcp(k, _):
        sl = pl.ds(k * 16, 16)
        den_v[sl] = jnp.float32(1.0) / (den_v[sl] + jnp.float32(1e-16))
        return _
    lax.fori_loop(0, 40, _rcp, None)

    _norm(0, 0)

    _zero_acc()
    plsc.subcore_barrier()
    _rows_stage(nchunk1, li1_v, ex1_v)
    plsc.subcore_barrier()
    _norm(_NH, 320)


def kernel(x, edge_index, Ws0, Wd0, atts0, attd0, b0, L0W, L0b,
           W1, atts1, attd1, b1, L1W, L1b, HW, Hb):
    src = edge_index[0].astype(jnp.int32)
    dst = edge_index[1].astype(jnp.int32)
    src_p = jnp.pad(src, (0, _EPAD - _E)).reshape(_NTILE, _NCHUNK, 128)
    dst_p = jnp.pad(dst, (0, _EPAD - _E)).reshape(_NTILE, _NCHUNK, 128)

    x_p = jnp.pad(x, ((0, _NP - _N), (0, 0)))
    wcat0 = jnp.concatenate([Ws0, Wd0, L0W], axis=1)
    skipb0 = (L0b + b0).reshape(1, _C)
    wcat1 = jnp.concatenate([W1, L1W], axis=1)
    skipb1 = (L1b + b1).reshape(1, _C)

    hsil, skip0, as0_3d, ad0_3d = _layer0_call(
        x_p, wcat0, skipb0, atts0.reshape(1, _C), attd0.reshape(1, _C))
    as0 = as0_3d.reshape(-1)[:_N]
    ad0 = ad0_3d.reshape(-1)[:_N]

    agg0 = _gat_edges(hsil, as0, ad0, src_p, dst_p)

    hpil, skip1, as1_3d, ad1_3d = _layer1_call(
        agg0, skip0, wcat1, skipb1, atts1.reshape(1, _C),
        attd1.reshape(1, _C))
    as1 = as1_3d.reshape(-1)[:_N]
    ad1 = ad1_3d.reshape(-1)[:_N]

    agg1 = _gat_edges(hpil, as1, ad1, src_p, dst_p)

    out_p = _head_call(agg1, skip1, HW, Hb.reshape(1, _HD))
    return out_p[:_N]


# speedup vs baseline: 1.0278x; 1.0278x over previous
import functools

import jax
import jax.numpy as jnp
from jax import lax
from jax.experimental import pallas as pl
from jax.experimental.pallas import tpu as pltpu
from jax.experimental.pallas import tpu_sc as plsc

_N = 10000
_E = 160000
_D = 256
_C = 256
_HD = 128

_R = 512
_GRID = 20
_NP = _R * _GRID

_NTILE = 16
_NCHUNK = 79
_EPT = _NCHUNK * 128
_EPAD = _EPT * _NTILE
_DENP = 10240


def _tc_layer0(x_ref, wcat_ref, skipb_ref, atts_ref, attd_ref,
               hsil_ref, skip_ref, as_ref, ad_ref):
    xb = x_ref[...]
    res = jnp.dot(xb, wcat_ref[...], preferred_element_type=jnp.float32)
    hs = res[:, :_C]
    hd = res[:, _C:2 * _C]
    sk = res[:, 2 * _C:] + skipb_ref[...]
    hsil_ref[...] = hs.reshape(_R, 2, 128).reshape(2 * _R, 128)
    skip_ref[...] = jnp.stack([sk[:, :128], sk[:, 128:]], axis=0)
    as_ref[...] = jnp.sum(hs * atts_ref[...], axis=1).reshape(1, 1, _R)
    ad_ref[...] = jnp.sum(hd * attd_ref[...], axis=1).reshape(1, 1, _R)


def _tc_layer1(agg_ref, skip_ref, wcat_ref, skipb_ref, atts_ref, attd_ref,
               hpil_ref, skip1_ref, as_ref, ad_ref):
    h2 = jax.nn.relu(agg_ref[...] + skip_ref[...])
    h = jnp.concatenate([h2[0], h2[1]], axis=1)
    res = jnp.dot(h, wcat_ref[...], preferred_element_type=jnp.float32)
    hp = res[:, :_C]
    sk1 = res[:, _C:] + skipb_ref[...]
    hpil_ref[...] = hp.reshape(_R, 2, 128).reshape(2 * _R, 128)
    skip1_ref[...] = jnp.stack([sk1[:, :128], sk1[:, 128:]], axis=0)
    as_ref[...] = jnp.sum(hp * atts_ref[...], axis=1).reshape(1, 1, _R)
    ad_ref[...] = jnp.sum(hp * attd_ref[...], axis=1).reshape(1, 1, _R)


def _tc_head(agg_ref, skip_ref, hw_ref, hb_ref, out_ref):
    h2 = agg_ref[...] + skip_ref[...]
    h = jnp.concatenate([h2[0], h2[1]], axis=1)
    out_ref[...] = (
        jnp.dot(h, hw_ref[...], preferred_element_type=jnp.float32)
        + hb_ref[...])


_f32 = jnp.float32

_layer0_call = pl.pallas_call(
    _tc_layer0,
    grid=(_GRID,),
    in_specs=[
        pl.BlockSpec((_R, _D), lambda i: (i, 0)),
        pl.BlockSpec((_D, 3 * _C), lambda i: (0, 0)),
        pl.BlockSpec((1, _C), lambda i: (0, 0)),
        pl.BlockSpec((1, _C), lambda i: (0, 0)),
        pl.BlockSpec((1, _C), lambda i: (0, 0)),
    ],
    out_specs=[
        pl.BlockSpec((2 * _R, 128), lambda i: (i, 0)),
        pl.BlockSpec((2, _R, 128), lambda i: (0, i, 0)),
        pl.BlockSpec((1, 1, _R), lambda i: (i, 0, 0)),
        pl.BlockSpec((1, 1, _R), lambda i: (i, 0, 0)),
    ],
    out_shape=[
        jax.ShapeDtypeStruct((2 * _NP, 128), _f32),
        jax.ShapeDtypeStruct((2, _NP, 128), _f32),
        jax.ShapeDtypeStruct((_GRID, 1, _R), _f32),
        jax.ShapeDtypeStruct((_GRID, 1, _R), _f32),
    ],
)

_layer1_call = pl.pallas_call(
    _tc_layer1,
    grid=(_GRID,),
    in_specs=[
        pl.BlockSpec((2, _R, 128), lambda i: (0, i, 0)),
        pl.BlockSpec((2, _R, 128), lambda i: (0, i, 0)),
        pl.BlockSpec((_C, 2 * _C), lambda i: (0, 0)),
        pl.BlockSpec((1, _C), lambda i: (0, 0)),
        pl.BlockSpec((1, _C), lambda i: (0, 0)),
        pl.BlockSpec((1, _C), lambda i: (0, 0)),
    ],
    out_specs=[
        pl.BlockSpec((2 * _R, 128), lambda i: (i, 0)),
        pl.BlockSpec((2, _R, 128), lambda i: (0, i, 0)),
        pl.BlockSpec((1, 1, _R), lambda i: (i, 0, 0)),
        pl.BlockSpec((1, 1, _R), lambda i: (i, 0, 0)),
    ],
    out_shape=[
        jax.ShapeDtypeStruct((2 * _NP, 128), _f32),
        jax.ShapeDtypeStruct((2, _NP, 128), _f32),
        jax.ShapeDtypeStruct((_GRID, 1, _R), _f32),
        jax.ShapeDtypeStruct((_GRID, 1, _R), _f32),
    ],
)

_head_call = pl.pallas_call(
    _tc_head,
    grid=(_GRID,),
    in_specs=[
        pl.BlockSpec((2, _R, 128), lambda i: (0, i, 0)),
        pl.BlockSpec((2, _R, 128), lambda i: (0, i, 0)),
        pl.BlockSpec((_C, _HD), lambda i: (0, 0)),
        pl.BlockSpec((1, _HD), lambda i: (0, 0)),
    ],
    out_specs=pl.BlockSpec((_R, _HD), lambda i: (i, 0)),
    out_shape=jax.ShapeDtypeStruct((_NP, _HD), _f32),
)


_mesh = plsc.VectorSubcoreMesh(core_axis_name="c", subcore_axis_name="s")

_NH = 5000
_NACC = 5008


@functools.partial(
    pl.kernel,
    out_type=jax.ShapeDtypeStruct((2, _NP, 128), _f32),
    mesh=_mesh,
    compiler_params=pltpu.CompilerParams(needs_layout_passes=False),
    scratch_types=[
        pltpu.VMEM((_NCHUNK, 128), jnp.int32),
        pltpu.VMEM((_NCHUNK, 128), jnp.int32),
        pltpu.VMEM((_NCHUNK, 128), jnp.int32),
        pltpu.VMEM((_NCHUNK, 128), _f32),
        pltpu.VMEM((_NCHUNK, 128), _f32),
        pltpu.VMEM((640,), _f32),
        pltpu.VMEM((128, 128), _f32),
        pltpu.VMEM((128, 128), _f32),
        pltpu.VMEM((128,), _f32),
        pltpu.VMEM((128,), _f32),
        pltpu.VMEM((128,), _f32),
        pltpu.VMEM((128,), _f32),
        pltpu.VMEM((128,), jnp.int32),
        pltpu.VMEM((128,), jnp.int32),
        pltpu.VMEM((128,), jnp.int32),
        pltpu.VMEM((128,), jnp.int32),
        pltpu.SemaphoreType.DMA,
        pltpu.SemaphoreType.DMA,
        pltpu.SemaphoreType.DMA,
        pltpu.SemaphoreType.DMA,
        pltpu.SemaphoreType.DMA,
        pltpu.SemaphoreType.DMA,
        pltpu.SemaphoreType.DMA,
        pltpu.VMEM_SHARED((_NACC, 128), _f32),
        pltpu.VMEM_SHARED((_DENP,), _f32),
    ],
)
def _gat_edges(hs_hbm, as_hbm, ad_hbm, src_hbm, dst_hbm, out_hbm,
               src_v, dst_v, li1_v, ex0_v, ex1_v, den_v, rows_a, rows_b,
               asg_a, asg_b, adg_a, adg_b, gst_a, gst_b, sst_a, sst_b,
               sem_ra, sem_rb, sem_sa, sem_sb, sem_da, sem_db, sem_dn,
               num_sh, den_sh):
    c = lax.axis_index("c")
    s = lax.axis_index("s")

    pltpu.sync_copy(src_hbm.at[s], src_v)
    pltpu.sync_copy(dst_hbm.at[s], dst_v)

    zs_f = jnp.zeros((16,), _f32)
    padp = jnp.full((16,), _NH << 15, jnp.int32)
    ebase = s * _EPT
    nch = jnp.where(s == _NTILE - 1, 5, 8)

    def _zero_den(k, _):
        den_v[pl.ds(k * 16, 16)] = zs_f
        return _
    lax.fori_loop(0, 40, _zero_den, None)
    pltpu.sync_copy(den_v, den_sh.at[pl.ds(s * 640, 640)])

    def _zero_acc():
        def _zero_row(r, _):
            for q in range(8):
                rows_a[r, pl.ds(q * 16, 16)] = zs_f
            return _
        lax.fori_loop(0, 40, _zero_row, None)

        def _zero_num(m, _):
            pltpu.sync_copy(rows_a.at[pl.ds(0, 40)],
                            num_sh.at[pl.ds(s * 320 + m * 40, 40)])
            return _
        lax.fori_loop(0, nch, _zero_num, None)

    _zero_acc()
    plsc.subcore_barrier()

    def _issue_ag(j, asg, adg, sem_s, sem_d):
        pltpu.async_copy(as_hbm.at[src_v.at[j]], asg, sem_s)
        pltpu.async_copy(ad_hbm.at[dst_v.at[j]], adg, sem_d)

    def _scan_chunk(j, cnts, asg, adg, sem_s, sem_d):
        pltpu.make_async_copy(as_hbm.at[src_v.at[j]], asg, sem_s).wait()
        pltpu.make_async_copy(ad_hbm.at[dst_v.at[j]], adg, sem_d).wait()

        def _grp(k, cnts):
            cnt0, cnt1 = cnts
            off = k * 16
            sl = pl.ds(off, 16)
            s16 = src_v[j, sl]
            d16 = dst_v[j, sl]
            al = asg[sl] + adg[sl]
            al = jnp.where(al >= 0.0, al, al * jnp.float32(0.2))
            ex = jnp.exp(al)
            gid = ebase + j * 128 + off + lax.iota(jnp.int32, 16)
            ex = jnp.where(gid < _E, ex, jnp.float32(0.0))
            rows_a[j, sl] = ex
            gi = s16 * 2 + c
            own0 = d16 < _NH
            i0 = own0.astype(jnp.int32)
            pos0 = cnt0 + plsc.cumsum(i0) - 1
            p0 = gi | (d16 << 15)
            plsc.store_scatter(src_v, [pos0 >> 7, pos0 & 127], p0, mask=own0)
            plsc.store_scatter(ex0_v, [pos0 >> 7, pos0 & 127], ex, mask=own0)
            own1 = jnp.logical_not(own0)
            pos1 = cnt1 + plsc.cumsum(1 - i0) - 1
            p1 = gi | ((d16 - _NH) << 15)
            plsc.store_scatter(li1_v, [pos1 >> 7, pos1 & 127], p1, mask=own1)
            plsc.store_scatter(ex1_v, [pos1 >> 7, pos1 & 127], ex, mask=own1)
            n0 = jnp.sum(i0)
            return (cnt0 + n0, cnt1 + (16 - n0))

        cnts = lax.fori_loop(0, 8, _grp, cnts)
        pltpu.async_copy(rows_a.at[j], den_sh.at[dst_v.at[j]], sem_dn,
                         add=True)
        return cnts

    _issue_ag(0, asg_a, adg_a, sem_sa, sem_da)

    def _scan_pair(pp, cnts):
        j0 = pp * 2
        _issue_ag(j0 + 1, asg_b, adg_b, sem_sb, sem_db)
        cnts = _scan_chunk(j0, cnts, asg_a, adg_a, sem_sa, sem_da)
        _issue_ag(j0 + 2, asg_a, adg_a, sem_sa, sem_da)
        cnts = _scan_chunk(j0 + 1, cnts, asg_b, adg_b, sem_sb, sem_db)
        return cnts

    cnts = lax.fori_loop(0, (_NCHUNK - 1) // 2, _scan_pair,
                         (jnp.int32(0), jnp.int32(0)))
    cnt0, cnt1 = _scan_chunk(_NCHUNK - 1, cnts, asg_a, adg_a, sem_sa, sem_da)

    def _drain_dn(j, _):
        pltpu.make_async_copy(rows_a.at[j], den_sh.at[dst_v.at[j]],
                              sem_dn).wait()
        return _
    lax.fori_loop(0, _NCHUNK, _drain_dn, None)

    def _pad_tail(cnt, li_ref, ex_ref):
        end = ((cnt + 127) >> 7) << 7
        for k in range(8):
            t = cnt + k * 16 + lax.iota(jnp.int32, 16)
            m = t < end
            rr = t >> 7
            cc = t & 127
            plsc.store_scatter(li_ref, [rr, cc], padp, mask=m)
            plsc.store_scatter(ex_ref, [rr, cc], zs_f, mask=m)
        return (cnt + 127) >> 7

    nchunk0 = _pad_tail(cnt0, src_v, ex0_v)
    nchunk1 = _pad_tail(cnt1, li1_v, ex1_v)

    def _rows_stage(nchunks, li_ref, ex_ref):
        def _unpack(jj, gst, sst):
            def _u(k, _):
                sl = pl.ds(k * 16, 16)
                p = li_ref[jj, sl]
                gst[sl] = p & 32767
                sst[sl] = lax.shift_right_logical(p, 15)
                return _
            lax.fori_loop(0, 8, _u, None)

        def _do(jj, rows_ref, gst, sst, sem):
            pltpu.make_async_copy(hs_hbm.at[gst], rows_ref, sem).wait()

            def _scale(r, _):
                e = plsc.load_gather(
                    ex_ref, [jnp.full((16,), jj, jnp.int32),
                             jnp.full((16,), r, jnp.int32)])
                for q in range(8):
                    sl = pl.ds(q * 16, 16)
                    rows_ref[r, sl] = rows_ref[r, sl] * e
                return _
            lax.fori_loop(0, 128, _scale, None, unroll=4)
            pltpu.sync_copy(rows_ref, num_sh.at[sst], add=True)

        @pl.when(nchunks > 0)
        def _():
            _unpack(0, gst_a, sst_a)
            pltpu.async_copy(hs_hbm.at[gst_a], rows_a, sem_ra)

        def _pairr(pp, _):
            j0 = pp * 2
            j1 = j0 + 1

            @pl.when(j1 < nchunks)
            def _():
                _unpack(j1, gst_b, sst_b)
                pltpu.async_copy(hs_hbm.at[gst_b], rows_b, sem_rb)
            _do(j0, rows_a, gst_a, sst_a, sem_ra)

            @pl.when(j1 < nchunks)
            def _():
                @pl.when(j1 + 1 < nchunks)
                def _():
                    _unpack(j1 + 1, gst_a, sst_a)
                    pltpu.async_copy(hs_hbm.at[gst_a], rows_a, sem_ra)
                _do(j1, rows_b, gst_b, sst_b, sem_rb)
            return _
        lax.fori_loop(0, (nchunks + 1) >> 1, _pairr, None)

    def _norm(nbase, dbase):
        def _body(m, _):
            l0 = s * 320 + m * 40
            pltpu.sync_copy(num_sh.at[pl.ds(l0, 40)], rows_a.at[pl.ds(0, 40)])

            def _scale_out(r, _):
                rc = plsc.load_gather(
                    den_v, [jnp.full((16,), dbase + m * 40 + r, jnp.int32)])
                for q in range(8):
                    sl = pl.ds(q * 16, 16)
                    rows_a[r, sl] = rows_a[r, sl] * rc
                return _
            lax.fori_loop(0, 40, _scale_out, None, unroll=4)
            pltpu.sync_copy(rows_a.at[pl.ds(0, 40)],
                            out_hbm.at[c, pl.ds(nbase + l0, 40)])
            return _
        lax.fori_loop(0, nch, _body, None)

    _rows_stage(nchunk0, src_v, ex0_v)
    plsc.subcore_barrier()

    pltpu.sync_copy(den_sh.at[pl.ds(s * 320, 320)], den_v.at[pl.ds(0, 320)])
    pltpu.sync_copy(den_sh.at[pl.ds(_NH + s * 320, 320)],
                    den_v.at[pl.ds(320, 320)])

    def _rcp(k, _):
        sl = pl.ds(k * 16, 16)
        den_v[sl] = jnp.float32(1.0) / (den_v[sl] + jnp.float32(1e-16))
        return _
    lax.fori_loop(0, 40, _rcp, None)

    _norm(0, 0)

    _zero_acc()
    plsc.subcore_barrier()
    _rows_stage(nchunk1, li1_v, ex1_v)
    plsc.subcore_barrier()
    _norm(_NH, 320)


def kernel(x, edge_index, Ws0, Wd0, atts0, attd0, b0, L0W, L0b,
           W1, atts1, attd1, b1, L1W, L1b, HW, Hb):
    src = edge_index[0].astype(jnp.int32)
    dst = edge_index[1].astype(jnp.int32)
    src_p = jnp.pad(src, (0, _EPAD - _E)).reshape(_NTILE, _NCHUNK, 128)
    dst_p = jnp.pad(dst, (0, _EPAD - _E)).reshape(_NTILE, _NCHUNK, 128)

    x_p = jnp.pad(x, ((0, _NP - _N), (0, 0)))
    wcat0 = jnp.concatenate([Ws0, Wd0, L0W], axis=1)
    skipb0 = (L0b + b0).reshape(1, _C)
    wcat1 = jnp.concatenate([W1, L1W], axis=1)
    skipb1 = (L1b + b1).reshape(1, _C)

    hsil, skip0, as0_3d, ad0_3d = _layer0_call(
        x_p, wcat0, skipb0, atts0.reshape(1, _C), attd0.reshape(1, _C))
    as0 = as0_3d.reshape(-1)[:_N]
    ad0 = ad0_3d.reshape(-1)[:_N]

    agg0 = _gat_edges(hsil, as0, ad0, src_p, dst_p)

    hpil, skip1, as1_3d, ad1_3d = _layer1_call(
        agg0, skip0, wcat1, skipb1, atts1.reshape(1, _C),
        attd1.reshape(1, _C))
    as1 = as1_3d.reshape(-1)[:_N]
    ad1 = ad1_3d.reshape(-1)[:_N]

    agg1 = _gat_edges(hpil, as1, ad1, src_p, dst_p)

    out_p = _head_call(agg1, skip1, HW, Hb.reshape(1, _HD))
    return out_p[:_N]
